# Initial kernel scaffold; baseline (speedup 1.0000x reference)
import jax
import jax.numpy as jnp
from jax.experimental import pallas as pl


def kernel(action, codebook):
    raise NotImplementedError("write your pallas kernel here")


# speedup vs baseline: 3.1144x; 3.1144x over previous
import functools

import jax
import jax.numpy as jnp
from jax.experimental import pallas as pl
from jax.experimental.pallas import tpu as pltpu

_B = 1048576
_D = 16
_K = 64
_BB = 8192


def _vq_block(a_ref, c_ref, zq_ref, idx_ref, msum_ref):
    a = a_ref[...]
    c = c_ref[...]
    x2 = jnp.sum(a * a, axis=1, keepdims=True)
    c2 = jnp.sum(c * c, axis=1)[None, :]
    ac = jax.lax.dot_general(a, c, (((1,), (1,)), ((), ())),
                             preferred_element_type=jnp.float32)
    d2 = x2 - 2.0 * ac + c2
    idx = jnp.argmin(d2, axis=1).astype(jnp.int32)
    idx_ref[...] = idx
    onehot = (jax.lax.broadcasted_iota(jnp.int32, (_BB, _K), 1)
              == idx[:, None]).astype(jnp.float32)
    zq_ref[...] = jax.lax.dot_general(onehot, c, (((1,), (0,)), ((), ())),
                                      preferred_element_type=jnp.float32)
    s = jnp.sum(jnp.min(d2, axis=1))

    @pl.when(pl.program_id(0) == 0)
    def _init():
        msum_ref[0, 0] = 0.0

    msum_ref[0, 0] += s


def kernel(action, codebook):
    n_blocks = _B // _BB
    zq, idx, msum = pl.pallas_call(
        _vq_block,
        grid=(n_blocks,),
        in_specs=[
            pl.BlockSpec((_BB, _D), lambda i: (i, 0)),
            pl.BlockSpec((_K, _D), lambda i: (0, 0)),
        ],
        out_specs=[
            pl.BlockSpec((_BB, _D), lambda i: (i, 0)),
            pl.BlockSpec((_BB,), lambda i: (i,)),
            pl.BlockSpec((1, 1), lambda i: (0, 0), memory_space=pltpu.SMEM),
        ],
        out_shape=[
            jax.ShapeDtypeStruct((_B, _D), jnp.float32),
            jax.ShapeDtypeStruct((_B,), jnp.int32),
            jax.ShapeDtypeStruct((1, 1), jnp.float32),
        ],
    )(action, codebook)
    mean_dist = msum[0, 0] / _B
    return (zq, idx, mean_dist)
